# Initial kernel scaffold; baseline (speedup 1.0000x reference)
import jax
import jax.numpy as jnp
from jax.experimental import pallas as pl


def kernel(x, W1, b1, W2, b2, edge_index):
    raise NotImplementedError("write your pallas kernel here")


# speedup vs baseline: 8.5777x; 8.5777x over previous
import functools

import jax
import jax.numpy as jnp
from jax import lax
from jax.experimental import pallas as pl
from jax.experimental.pallas import tpu as pltpu
from jax.experimental.pallas import tpu_sc as plsc

NC = 2
NS = 16
NTILES = NC * NS
LANES = 16
CHUNK = 128
DEGW = 16
ZROWS = 64


def _sc_mesh():
    return plsc.VectorSubcoreMesh(
        core_axis_name="c", subcore_axis_name="s", num_cores=NC, num_subcores=NS
    )


def _make_deg_kernel(n_pad, cpt):
    rows_per_tile = n_pad // NS

    @functools.partial(
        pl.kernel,
        out_type=jax.ShapeDtypeStruct((NC, n_pad, DEGW), jnp.float32),
        mesh=_sc_mesh(),
        scratch_types=[
            pltpu.VMEM((cpt, CHUNK), jnp.int32),
            pltpu.VMEM((CHUNK, DEGW), jnp.float32),
            pltpu.VMEM((ZROWS, DEGW), jnp.float32),
            pltpu.VMEM_SHARED((n_pad, DEGW), jnp.float32),
        ],
    )
    def deg_kernel(row_hbm, out_hbm, idxb, onesb, zb, accum):
        cid = lax.axis_index("c")
        sid = lax.axis_index("s")
        wid = cid * NS + sid
        lane = lax.iota(jnp.int32, LANES)
        onevec = jnp.where(lane == 0, 1.0, 0.0).astype(jnp.float32)
        zvec = jnp.zeros((LANES,), jnp.float32)

        def fill_ones(r, carry):
            onesb[r, :] = onevec
            return carry

        lax.fori_loop(0, CHUNK, fill_ones, 0)

        def fill_z(r, carry):
            zb[r, :] = zvec
            return carry

        lax.fori_loop(0, ZROWS, fill_z, 0)

        def zero_accum(j, carry):
            pltpu.sync_copy(zb, accum.at[pl.ds(sid * rows_per_tile + j * ZROWS, ZROWS)])
            return carry

        lax.fori_loop(0, rows_per_tile // ZROWS, zero_accum, 0)
        plsc.subcore_barrier()

        pltpu.sync_copy(row_hbm.at[pl.ds(wid * cpt, cpt)], idxb)

        def step(j, carry):
            pltpu.sync_copy(onesb, accum.at[idxb.at[j]], add=True)
            return carry

        lax.fori_loop(0, cpt, step, 0)
        plsc.subcore_barrier()
        pltpu.sync_copy(
            accum.at[pl.ds(sid * rows_per_tile, rows_per_tile)],
            out_hbm.at[cid, pl.ds(sid * rows_per_tile, rows_per_tile)],
        )

    return deg_kernel


def _make_agg_kernel(n, d, n_pad, cpt):
    rows_per_tile = n_pad // NS

    @functools.partial(
        pl.kernel,
        out_type=jax.ShapeDtypeStruct((NC, n_pad, d), jnp.float32),
        mesh=_sc_mesh(),
        scratch_types=[
            pltpu.VMEM((cpt, CHUNK), jnp.int32),
            pltpu.VMEM((cpt, CHUNK), jnp.int32),
            pltpu.VMEM((CHUNK, d), jnp.float32),
            pltpu.VMEM((ZROWS, d), jnp.float32),
            pltpu.VMEM_SHARED((n_pad, d), jnp.float32),
            pltpu.SemaphoreType.DMA,
        ],
    )
    def agg_kernel(g_hbm, col_hbm, row_hbm, out_hbm, cib, rib, gbuf, zb, accum, sem):
        cid = lax.axis_index("c")
        sid = lax.axis_index("s")
        wid = cid * NS + sid
        zvec = jnp.zeros((LANES,), jnp.float32)

        def fill_z(r, carry):
            for cc in range(d // LANES):
                zb[r, pl.ds(cc * LANES, LANES)] = zvec
            return carry

        lax.fori_loop(0, ZROWS, fill_z, 0)

        def zero_accum(j, carry):
            pltpu.sync_copy(zb, accum.at[pl.ds(sid * rows_per_tile + j * ZROWS, ZROWS)])
            return carry

        lax.fori_loop(0, rows_per_tile // ZROWS, zero_accum, 0)
        plsc.subcore_barrier()

        pltpu.sync_copy(col_hbm.at[pl.ds(wid * cpt, cpt)], cib)
        pltpu.sync_copy(row_hbm.at[pl.ds(wid * cpt, cpt)], rib)

        def step(j, carry):
            pltpu.async_copy(g_hbm.at[cib.at[j]], gbuf, sem).wait()
            pltpu.sync_copy(gbuf, accum.at[rib.at[j]], add=True)
            return carry

        lax.fori_loop(0, cpt, step, 0)
        plsc.subcore_barrier()
        pltpu.sync_copy(
            accum.at[pl.ds(sid * rows_per_tile, rows_per_tile)],
            out_hbm.at[cid, pl.ds(sid * rows_per_tile, rows_per_tile)],
        )

    return agg_kernel


def _dinv(degp_ref):
    return lax.rsqrt(degp_ref[0, :, 0:1] + degp_ref[1, :, 0:1] + 1.0)


def _elu(t):
    return jnp.where(t > 0, t, jnp.exp(t) - 1.0)


def _tc1_body(x_ref, w_ref, degp_ref, o_ref):
    h = jnp.dot(x_ref[...], w_ref[...], preferred_element_type=jnp.float32)
    o_ref[...] = h * _dinv(degp_ref)


def _tc2_body(s_ref, g_ref, degp_ref, b_ref, w_ref, o_ref):
    dinv = _dinv(degp_ref)
    t = dinv * (s_ref[0] + s_ref[1] + g_ref[...]) + b_ref[...]
    h2 = jnp.dot(_elu(t), w_ref[...], preferred_element_type=jnp.float32)
    o_ref[...] = h2 * dinv


def _tc3_body(s_ref, g_ref, degp_ref, b_ref, o_ref):
    t = _dinv(degp_ref) * (s_ref[0] + s_ref[1] + g_ref[...]) + b_ref[...]
    o_ref[...] = _elu(t)


def _run_tc1(x, w, degp, bm):
    n, d = x.shape
    return pl.pallas_call(
        _tc1_body,
        grid=(n // bm,),
        in_specs=[
            pl.BlockSpec((bm, d), lambda i: (i, 0)),
            pl.BlockSpec((d, d), lambda i: (0, 0)),
            pl.BlockSpec((NC, bm, DEGW), lambda i: (0, i, 0)),
        ],
        out_specs=pl.BlockSpec((bm, d), lambda i: (i, 0)),
        out_shape=jax.ShapeDtypeStruct((n, d), jnp.float32),
    )(x, w, degp)


def _run_tc2(s, g, degp, b, w, bm):
    n, d = g.shape
    return pl.pallas_call(
        _tc2_body,
        grid=(n // bm,),
        in_specs=[
            pl.BlockSpec((NC, bm, d), lambda i: (0, i, 0)),
            pl.BlockSpec((bm, d), lambda i: (i, 0)),
            pl.BlockSpec((NC, bm, DEGW), lambda i: (0, i, 0)),
            pl.BlockSpec((1, d), lambda i: (0, 0)),
            pl.BlockSpec((d, d), lambda i: (0, 0)),
        ],
        out_specs=pl.BlockSpec((bm, d), lambda i: (i, 0)),
        out_shape=jax.ShapeDtypeStruct((n, d), jnp.float32),
    )(s, g, degp, b, w)


def _run_tc3(s, g, degp, b, bm):
    n, d = g.shape
    return pl.pallas_call(
        _tc3_body,
        grid=(n // bm,),
        in_specs=[
            pl.BlockSpec((NC, bm, d), lambda i: (0, i, 0)),
            pl.BlockSpec((bm, d), lambda i: (i, 0)),
            pl.BlockSpec((NC, bm, DEGW), lambda i: (0, i, 0)),
            pl.BlockSpec((1, d), lambda i: (0, 0)),
        ],
        out_specs=pl.BlockSpec((bm, d), lambda i: (i, 0)),
        out_shape=jax.ShapeDtypeStruct((n, d), jnp.float32),
    )(s, g, degp, b)


def kernel(x, W1, b1, W2, b2, edge_index):
    n, d = x.shape
    e = edge_index.shape[1]
    col = edge_index[0]
    row = edge_index[1]

    cpt = -(-e // (NTILES * CHUNK))
    cpt = -(-cpt // 8) * 8
    e_pad = cpt * CHUNK * NTILES
    pad = e_pad - e
    align = NS * ZROWS
    n_pad = -(-n // align) * align
    dummy = n + (jnp.arange(pad, dtype=jnp.int32) % (n_pad - n))
    col_p = jnp.concatenate([col, jnp.zeros((pad,), jnp.int32)]).reshape(-1, CHUNK)
    row_p = jnp.concatenate([row, dummy]).reshape(-1, CHUNK)

    degp = _make_deg_kernel(n_pad, cpt)(row_p)
    agg = _make_agg_kernel(n, d, n_pad, cpt)

    bm = 1000
    b1r = b1.reshape(1, d)
    b2r = b2.reshape(1, d)

    g1 = _run_tc1(x, W1, degp, bm)
    s1 = agg(g1, col_p, row_p)
    g2 = _run_tc2(s1, g1, degp, b1r, W2, bm)
    s2 = agg(g2, col_p, row_p)
    return _run_tc3(s2, g2, degp, b2r, bm)


# speedup vs baseline: 9.5745x; 1.1162x over previous
import functools

import jax
import jax.numpy as jnp
from jax import lax
from jax.experimental import pallas as pl
from jax.experimental.pallas import tpu as pltpu
from jax.experimental.pallas import tpu_sc as plsc

NC = 2
NS = 16
NTILES = NC * NS
LANES = 16
CHUNK = 128
DEGW = 16
ZROWS = 64


def _sc_mesh():
    return plsc.VectorSubcoreMesh(
        core_axis_name="c", subcore_axis_name="s", num_cores=NC, num_subcores=NS
    )


def _make_deg_kernel(n_pad, cpt):
    rows_per_tile = n_pad // NS

    @functools.partial(
        pl.kernel,
        out_type=jax.ShapeDtypeStruct((NC, n_pad, DEGW), jnp.float32),
        mesh=_sc_mesh(),
        scratch_types=[
            pltpu.VMEM((cpt, CHUNK), jnp.int32),
            pltpu.VMEM((CHUNK, DEGW), jnp.float32),
            pltpu.VMEM((ZROWS, DEGW), jnp.float32),
            pltpu.VMEM_SHARED((n_pad, DEGW), jnp.float32),
        ],
    )
    def deg_kernel(row_hbm, out_hbm, idxb, onesb, zb, accum):
        cid = lax.axis_index("c")
        sid = lax.axis_index("s")
        wid = cid * NS + sid
        lane = lax.iota(jnp.int32, LANES)
        onevec = jnp.where(lane == 0, 1.0, 0.0).astype(jnp.float32)
        zvec = jnp.zeros((LANES,), jnp.float32)

        def fill_ones(r, carry):
            onesb[r, :] = onevec
            return carry

        lax.fori_loop(0, CHUNK, fill_ones, 0)

        def fill_z(r, carry):
            zb[r, :] = zvec
            return carry

        lax.fori_loop(0, ZROWS, fill_z, 0)

        def zero_accum(j, carry):
            pltpu.sync_copy(zb, accum.at[pl.ds(sid * rows_per_tile + j * ZROWS, ZROWS)])
            return carry

        lax.fori_loop(0, rows_per_tile // ZROWS, zero_accum, 0)
        plsc.subcore_barrier()

        pltpu.sync_copy(row_hbm.at[pl.ds(wid * cpt, cpt)], idxb)

        def step(j, carry):
            pltpu.sync_copy(onesb, accum.at[idxb.at[j]], add=True)
            return carry

        lax.fori_loop(0, cpt, step, 0)
        plsc.subcore_barrier()
        pltpu.sync_copy(
            accum.at[pl.ds(sid * rows_per_tile, rows_per_tile)],
            out_hbm.at[cid, pl.ds(sid * rows_per_tile, rows_per_tile)],
        )

    return deg_kernel


NBUF = 2
SPC = 8


def _make_agg_kernel(n, d, n_pad, cpt):
    rows_per_tile = n_pad // NS
    assert cpt % (2 * SPC) == 0 and rows_per_tile % CHUNK == 0
    nstripes = cpt // SPC

    @functools.partial(
        pl.kernel,
        out_type=jax.ShapeDtypeStruct((NC, n_pad, d), jnp.float32),
        mesh=_sc_mesh(),
        scratch_types=[
            pltpu.VMEM((2, SPC, CHUNK), jnp.int32),
            pltpu.VMEM((2, SPC, CHUNK), jnp.int32),
            pltpu.VMEM((NBUF, CHUNK, d), jnp.float32),
            pltpu.VMEM_SHARED((n_pad, d), jnp.float32),
            [pltpu.SemaphoreType.DMA] * NBUF,
            [pltpu.SemaphoreType.DMA] * 2,
        ],
    )
    def agg_kernel(g_hbm, col_hbm, row_hbm, out_hbm, cib, rib, gbuf, accum,
                   semg, semi):
        cid = lax.axis_index("c")
        sid = lax.axis_index("s")
        wid = cid * NS + sid
        zvec = jnp.zeros((LANES,), jnp.float32)

        def fill_z(r, carry):
            for cc in range(d // LANES):
                gbuf[0, r, pl.ds(cc * LANES, LANES)] = zvec
            return carry

        lax.fori_loop(0, CHUNK, fill_z, 0)

        def zero_accum(j, carry):
            pltpu.sync_copy(
                gbuf.at[0], accum.at[pl.ds(sid * rows_per_tile + j * CHUNK, CHUNK)]
            )
            return carry

        lax.fori_loop(0, rows_per_tile // CHUNK, zero_accum, 0)
        plsc.subcore_barrier()

        def load_stripe(buf, s):
            base = wid * cpt + s * SPC
            pltpu.async_copy(col_hbm.at[pl.ds(base, SPC)], cib.at[buf], semi[buf])
            pltpu.async_copy(row_hbm.at[pl.ds(base, SPC)], rib.at[buf], semi[buf])

        def wait_stripe(buf):
            pltpu.make_async_copy(col_hbm.at[pl.ds(0, SPC)], cib.at[buf], semi[buf]).wait()
            pltpu.make_async_copy(row_hbm.at[pl.ds(0, SPC)], rib.at[buf], semi[buf]).wait()

        def start_gather(p, r, b):
            pltpu.async_copy(g_hbm.at[cib.at[p, r]], gbuf.at[b], semg[b])

        def wait_gather(b):
            pltpu.make_async_copy(g_hbm.at[cib.at[0, 0]], gbuf.at[b], semg[b]).wait()

        load_stripe(0, 0)
        wait_stripe(0)
        load_stripe(1, 1)
        start_gather(0, 0, 0)
        start_gather(0, 1, 1)

        nouter = nstripes // 2

        def outer(o, carry):
            for k in range(2 * SPC):
                b = k % 2
                p, r = k // SPC, k % SPC
                wait_gather(b)
                pltpu.sync_copy(gbuf.at[b], accum.at[rib.at[p, r]], add=True)
                if k == SPC - 1:
                    @pl.when(o + 1 < nouter)
                    def _():
                        load_stripe(0, 2 * (o + 1))

                kk = k + 2
                if kk < SPC:
                    start_gather(0, kk, b)
                elif kk == SPC:
                    wait_stripe(1)
                    start_gather(1, 0, b)
                elif kk < 2 * SPC:
                    start_gather(1, kk - SPC, b)
                else:
                    @pl.when(o + 1 < nouter)
                    def _():
                        if kk == 2 * SPC:
                            wait_stripe(0)
                        start_gather(0, kk - 2 * SPC, b)

            @pl.when(o + 1 < nouter)
            def _():
                load_stripe(1, 2 * (o + 1) + 1)

            return carry

        lax.fori_loop(0, nouter, outer, 0)
        plsc.subcore_barrier()
        pltpu.sync_copy(
            accum.at[pl.ds(sid * rows_per_tile, rows_per_tile)],
            out_hbm.at[cid, pl.ds(sid * rows_per_tile, rows_per_tile)],
        )

    return agg_kernel


def _dinv(degp_ref):
    return lax.rsqrt(degp_ref[0, :, 0:1] + degp_ref[1, :, 0:1] + 1.0)


def _elu(t):
    return jnp.where(t > 0, t, jnp.exp(t) - 1.0)


def _tc1_body(x_ref, w_ref, degp_ref, o_ref):
    h = jnp.dot(x_ref[...], w_ref[...], preferred_element_type=jnp.float32)
    o_ref[...] = h * _dinv(degp_ref)


def _tc2_body(s_ref, g_ref, degp_ref, b_ref, w_ref, o_ref):
    dinv = _dinv(degp_ref)
    t = dinv * (s_ref[0] + s_ref[1] + g_ref[...]) + b_ref[...]
    h2 = jnp.dot(_elu(t), w_ref[...], preferred_element_type=jnp.float32)
    o_ref[...] = h2 * dinv


def _tc3_body(s_ref, g_ref, degp_ref, b_ref, o_ref):
    t = _dinv(degp_ref) * (s_ref[0] + s_ref[1] + g_ref[...]) + b_ref[...]
    o_ref[...] = _elu(t)


def _run_tc1(x, w, degp, bm):
    n, d = x.shape
    return pl.pallas_call(
        _tc1_body,
        grid=(n // bm,),
        in_specs=[
            pl.BlockSpec((bm, d), lambda i: (i, 0)),
            pl.BlockSpec((d, d), lambda i: (0, 0)),
            pl.BlockSpec((NC, bm, DEGW), lambda i: (0, i, 0)),
        ],
        out_specs=pl.BlockSpec((bm, d), lambda i: (i, 0)),
        out_shape=jax.ShapeDtypeStruct((n, d), jnp.float32),
    )(x, w, degp)


def _run_tc2(s, g, degp, b, w, bm):
    n, d = g.shape
    return pl.pallas_call(
        _tc2_body,
        grid=(n // bm,),
        in_specs=[
            pl.BlockSpec((NC, bm, d), lambda i: (0, i, 0)),
            pl.BlockSpec((bm, d), lambda i: (i, 0)),
            pl.BlockSpec((NC, bm, DEGW), lambda i: (0, i, 0)),
            pl.BlockSpec((1, d), lambda i: (0, 0)),
            pl.BlockSpec((d, d), lambda i: (0, 0)),
        ],
        out_specs=pl.BlockSpec((bm, d), lambda i: (i, 0)),
        out_shape=jax.ShapeDtypeStruct((n, d), jnp.float32),
    )(s, g, degp, b, w)


def _run_tc3(s, g, degp, b, bm):
    n, d = g.shape
    return pl.pallas_call(
        _tc3_body,
        grid=(n // bm,),
        in_specs=[
            pl.BlockSpec((NC, bm, d), lambda i: (0, i, 0)),
            pl.BlockSpec((bm, d), lambda i: (i, 0)),
            pl.BlockSpec((NC, bm, DEGW), lambda i: (0, i, 0)),
            pl.BlockSpec((1, d), lambda i: (0, 0)),
        ],
        out_specs=pl.BlockSpec((bm, d), lambda i: (i, 0)),
        out_shape=jax.ShapeDtypeStruct((n, d), jnp.float32),
    )(s, g, degp, b)


def kernel(x, W1, b1, W2, b2, edge_index):
    n, d = x.shape
    e = edge_index.shape[1]
    col = edge_index[0]
    row = edge_index[1]

    cpt = -(-e // (NTILES * CHUNK))
    cpt = -(-cpt // 8) * 8
    e_pad = cpt * CHUNK * NTILES
    pad = e_pad - e
    align = NS * ZROWS
    n_pad = -(-n // align) * align
    dummy = n + (jnp.arange(pad, dtype=jnp.int32) % (n_pad - n))
    col_p = jnp.concatenate([col, jnp.zeros((pad,), jnp.int32)]).reshape(-1, CHUNK)
    row_p = jnp.concatenate([row, dummy]).reshape(-1, CHUNK)

    degp = _make_deg_kernel(n_pad, cpt)(row_p)
    agg = _make_agg_kernel(n, d, n_pad, cpt)

    bm = 1000
    b1r = b1.reshape(1, d)
    b2r = b2.reshape(1, d)

    g1 = _run_tc1(x, W1, degp, bm)
    s1 = agg(g1, col_p, row_p)
    g2 = _run_tc2(s1, g1, degp, b1r, W2, bm)
    s2 = agg(g2, col_p, row_p)
    return _run_tc3(s2, g2, degp, b2r, bm)
